# baseline (speedup 1.0000x reference)
import functools

import jax
import jax.numpy as jnp
from jax import lax
from jax.experimental import pallas as pl
from jax.experimental.pallas import tpu as pltpu
from jax.experimental.pallas import tpu_sc as plsc

_NUM_CLASSES = 1000000
_D = 64
_B = 16384
_LAMBDA_C = 0.5

_NC = 2
_NS = 16
_NW = _NC * _NS
_BPW = _B // _NW
_W = 16
_NWAVE = _BPW // _W
_LANES = 16
_SCALE = _LAMBDA_C / (2.0 * _B)


def _body(idx_hbm, feat_hbm, c2_hbm, out_hbm,
          idx_v, rows_v, feat_v, res_v, gsem, fsem):
    wid = lax.axis_index("s") * _NC + lax.axis_index("c")
    base = wid * _BPW

    pltpu.sync_copy(idx_hbm.at[pl.ds(base, _BPW)], idx_v)
    fbase = pl.multiple_of(base // 2, _BPW // 2)
    fcopy = pltpu.async_copy(
        feat_hbm.at[pl.ds(fbase, _BPW // 2)], feat_v, fsem)

    def fire(w, buf):
        tv = idx_v[pl.ds(w * _W, _W)]
        pair = lax.shift_right_logical(tv, 1)
        pltpu.async_copy(c2_hbm.at[pair], rows_v.at[buf], gsem.at[buf])
        return tv

    def drain(buf):
        pltpu.make_async_copy(
            c2_hbm.at[pl.ds(0, _W)],
            rows_v.at[buf],
            gsem.at[buf],
        ).wait()

    def compute(w, buf, tv, acc):
        for k in range(_W):
            b = w * _W + k
            w_hi = (tv[k] & 1).astype(jnp.float32)
            w_lo = 1.0 - w_hi
            for c in range(_D // _LANES):
                f = feat_v[b // 2, pl.ds((b % 2) * _D + c * _LANES, _LANES)]
                g_lo = rows_v[buf, k, pl.ds(c * _LANES, _LANES)]
                g_hi = rows_v[buf, k, pl.ds(_D + c * _LANES, _LANES)]
                d = f - (g_lo * w_lo + g_hi * w_hi)
                acc = acc + d * d
        return acc

    tv0 = fire(0, 0)
    fcopy.wait()

    def wave_step(w, carry):
        tvp, acc = carry
        buf = lax.rem(w, 2)
        tv = fire(w, buf)
        pbuf = lax.rem(w + 1, 2)
        drain(pbuf)
        acc = compute(w - 1, pbuf, tvp, acc)
        return tv, acc

    tvl, acc = lax.fori_loop(
        1, _NWAVE, wave_step, (tv0, jnp.zeros((_LANES,), jnp.float32)))
    last = _NWAVE - 1
    lbuf = last % 2
    drain(lbuf)
    acc = compute(last, lbuf, tvl, acc)

    res_v[...] = acc * _SCALE
    pltpu.sync_copy(res_v, out_hbm.at[wid])


@jax.jit
def kernel(features, targets, centers):
    idx = targets.astype(jnp.int32)
    centers2 = jnp.reshape(centers, (_NUM_CLASSES // 2, 2 * _D))
    feat2 = jnp.reshape(features, (_B // 2, 2 * _D))
    run = functools.partial(
        pl.kernel,
        out_type=jax.ShapeDtypeStruct((_NW, _LANES), jnp.float32),
        mesh=plsc.VectorSubcoreMesh(core_axis_name="c", subcore_axis_name="s"),
        scratch_types=[
            pltpu.VMEM((_BPW,), jnp.int32),
            pltpu.VMEM((2, _W, 2 * _D), jnp.float32),
            pltpu.VMEM((_BPW // 2, 2 * _D), jnp.float32),
            pltpu.VMEM((_LANES,), jnp.float32),
            pltpu.SemaphoreType.DMA((2,)),
            pltpu.SemaphoreType.DMA,
        ],
    )(_body)
    partials = run(idx, feat2, centers2)
    return jnp.sum(partials)


# speedup vs baseline: 2.0539x; 2.0539x over previous
import functools

import jax
import jax.numpy as jnp
from jax import lax
from jax.experimental import pallas as pl
from jax.experimental.pallas import tpu as pltpu
from jax.experimental.pallas import tpu_sc as plsc

_NUM_CLASSES = 1000000
_D = 64
_B = 16384
_LAMBDA_C = 0.5

_NC = 2
_NS = 16
_NW = _NC * _NS
_BPW = _B // _NW
_W = 16
_NWAVE = _BPW // _W
_LANES = 16
_SCALE = _LAMBDA_C / (2.0 * _B)

_CB = 2048
_CGRID = (_NUM_CLASSES + 2 * _CB - 1) // (2 * _CB)
_CROWS = _CGRID * _CB
_FROWS = _B // 2


def _tpose_body(x1_ref, x2_ref, out_ref):
    z = jnp.concatenate([x1_ref[...], x2_ref[...]], axis=0)
    r = lax.broadcasted_iota(jnp.int32, (2 * _D, 2 * _D), 0)
    c = lax.broadcasted_iota(jnp.int32, (2 * _D, 2 * _D), 1)
    ident = (r == c).astype(jnp.float32)
    out_ref[...] = lax.dot_general(
        z, ident, (((0,), (0,)), ((), ())),
        preferred_element_type=jnp.float32)


def _pack(ct, ngrid, nrows):
    nblk = pl.cdiv(ct.shape[1], _CB)
    return pl.pallas_call(
        _tpose_body,
        grid=(ngrid,),
        in_specs=[
            pl.BlockSpec((_D, _CB), lambda i: (0, 2 * i)),
            pl.BlockSpec((_D, _CB), lambda i: (0, jnp.minimum(2 * i + 1,
                                                              nblk - 1))),
        ],
        out_specs=pl.BlockSpec((_CB, 2 * _D), lambda i: (i, 0)),
        out_shape=jax.ShapeDtypeStruct((nrows, 2 * _D), jnp.float32),
    )(ct, ct)


def _body(idx_hbm, feat_hbm, c2_hbm, out_hbm,
          idx_v, rows_v, feat_v, res_v, gsem, fsem):
    wid = lax.axis_index("s") * _NC + lax.axis_index("c")
    base = wid * _BPW

    pltpu.sync_copy(idx_hbm.at[pl.ds(base, _BPW)], idx_v)
    frow = pl.multiple_of(
        lax.shift_right_logical(wid, 3) * 2048 + (wid & 3) * _BPW, _BPW)
    fhalf = lax.shift_right_logical(wid, 2) & 1
    fcopy = pltpu.async_copy(feat_hbm.at[pl.ds(frow, _BPW)], feat_v, fsem)

    def fire(w, buf):
        tv = idx_v[pl.ds(w * _W, _W)]
        row = jnp.bitwise_or(
            lax.shift_left(lax.shift_right_logical(tv, 12), 11),
            tv & 2047)
        pltpu.async_copy(c2_hbm.at[row], rows_v.at[buf], gsem.at[buf])
        return tv

    def drain(buf):
        pltpu.make_async_copy(
            c2_hbm.at[pl.ds(0, _W)],
            rows_v.at[buf],
            gsem.at[buf],
        ).wait()

    def compute(w, buf, tv, acc):
        for k in range(_W):
            w_hi = ((lax.shift_right_logical(tv[k], 11)) & 1).astype(
                jnp.float32)
            w_lo = 1.0 - w_hi
            foff = fhalf * _D
            for c in range(_D // _LANES):
                f = feat_v[w * _W + k, pl.ds(foff + c * _LANES, _LANES)]
                g_lo = rows_v[buf, k, pl.ds(c * _LANES, _LANES)]
                g_hi = rows_v[buf, k, pl.ds(_D + c * _LANES, _LANES)]
                d = f - (g_lo * w_lo + g_hi * w_hi)
                acc = acc + d * d
        return acc

    tv0 = fire(0, 0)
    fcopy.wait()

    def wave_step(w, carry):
        tvp, acc = carry
        buf = lax.rem(w, 2)
        tv = fire(w, buf)
        pbuf = lax.rem(w + 1, 2)
        drain(pbuf)
        acc = compute(w - 1, pbuf, tvp, acc)
        return tv, acc

    tvl, acc = lax.fori_loop(
        1, _NWAVE, wave_step, (tv0, jnp.zeros((_LANES,), jnp.float32)))
    last = _NWAVE - 1
    lbuf = last % 2
    drain(lbuf)
    acc = compute(last, lbuf, tvl, acc)

    res_v[...] = acc * _SCALE
    pltpu.sync_copy(res_v, out_hbm.at[wid])


@jax.jit
def kernel(features, targets, centers):
    idx = targets.astype(jnp.int32)
    centers2 = _pack(jnp.transpose(centers), _CGRID, _CROWS)
    feat2 = _pack(jnp.transpose(features), _B // (2 * _CB), _FROWS)
    run = functools.partial(
        pl.kernel,
        out_type=jax.ShapeDtypeStruct((_NW, _LANES), jnp.float32),
        mesh=plsc.VectorSubcoreMesh(core_axis_name="c", subcore_axis_name="s"),
        scratch_types=[
            pltpu.VMEM((_BPW,), jnp.int32),
            pltpu.VMEM((2, _W, 2 * _D), jnp.float32),
            pltpu.VMEM((_BPW, 2 * _D), jnp.float32),
            pltpu.VMEM((_LANES,), jnp.float32),
            pltpu.SemaphoreType.DMA((2,)),
            pltpu.SemaphoreType.DMA,
        ],
    )(_body)
    partials = run(idx, feat2, centers2)
    return jnp.sum(partials)


# speedup vs baseline: 2.7523x; 1.3400x over previous
import functools

import jax
import jax.numpy as jnp
from jax import lax
from jax.experimental import pallas as pl
from jax.experimental.pallas import tpu as pltpu
from jax.experimental.pallas import tpu_sc as plsc

_NUM_CLASSES = 1000000
_D = 64
_B = 16384
_LAMBDA_C = 0.5

_NC = 2
_NS = 16
_NW = _NC * _NS
_BPW = _B // _NW
_W = 16
_NWAVE = _BPW // _W
_LANES = 16
_SCALE = _LAMBDA_C / (2.0 * _B)

_CB = 4096
_CSH = 12
_CGRID = (_NUM_CLASSES + 2 * _CB - 1) // (2 * _CB)
_CROWS = _CGRID * _CB
_FROWS = _B // 2


def _tpose_body(x1_ref, x2_ref, out_ref):
    z = jnp.concatenate([x1_ref[...], x2_ref[...]], axis=0)
    r = lax.broadcasted_iota(jnp.int32, (2 * _D, 2 * _D), 0)
    c = lax.broadcasted_iota(jnp.int32, (2 * _D, 2 * _D), 1)
    ident = (r == c).astype(jnp.float32)
    out_ref[...] = lax.dot_general(
        z, ident, (((0,), (0,)), ((), ())),
        preferred_element_type=jnp.float32)


def _pack(ct, ngrid, nrows):
    nblk = pl.cdiv(ct.shape[1], _CB)
    return pl.pallas_call(
        _tpose_body,
        grid=(ngrid,),
        in_specs=[
            pl.BlockSpec((_D, _CB), lambda i: (0, 2 * i)),
            pl.BlockSpec((_D, _CB), lambda i: (0, jnp.minimum(2 * i + 1,
                                                              nblk - 1))),
        ],
        out_specs=pl.BlockSpec((_CB, 2 * _D), lambda i: (i, 0)),
        out_shape=jax.ShapeDtypeStruct((nrows, 2 * _D), jnp.float32),
    )(ct, ct)


def _body(idx_hbm, feat_hbm, c2_hbm, out_hbm,
          idx_v, rows_v, feat_v, res_v, gsem, fsem):
    wid = lax.axis_index("s") * _NC + lax.axis_index("c")
    base = wid * _BPW

    pltpu.sync_copy(idx_hbm.at[pl.ds(base, _BPW)], idx_v)
    nw_grp = 2 * _CB // _BPW
    frow = pl.multiple_of(
        (wid // nw_grp) * _CB + (wid % (nw_grp // 2)) * _BPW, _BPW)
    fhalf = (wid // (nw_grp // 2)) & 1
    fcopy = pltpu.async_copy(feat_hbm.at[pl.ds(frow, _BPW)], feat_v, fsem)

    def fire(w, buf):
        tv = idx_v[pl.ds(w * _W, _W)]
        row = jnp.bitwise_or(
            lax.shift_left(lax.shift_right_logical(tv, _CSH + 1), _CSH),
            tv & (_CB - 1))
        pltpu.async_copy(c2_hbm.at[row], rows_v.at[buf], gsem.at[buf])
        return tv

    def drain(buf):
        pltpu.make_async_copy(
            c2_hbm.at[pl.ds(0, _W)],
            rows_v.at[buf],
            gsem.at[buf],
        ).wait()

    def compute(w, buf, tv, acc):
        for k in range(_W):
            w_hi = ((lax.shift_right_logical(tv[k], _CSH)) & 1).astype(
                jnp.float32)
            w_lo = 1.0 - w_hi
            foff = fhalf * _D
            for c in range(_D // _LANES):
                f = feat_v[w * _W + k, pl.ds(foff + c * _LANES, _LANES)]
                g_lo = rows_v[buf, k, pl.ds(c * _LANES, _LANES)]
                g_hi = rows_v[buf, k, pl.ds(_D + c * _LANES, _LANES)]
                d = f - (g_lo * w_lo + g_hi * w_hi)
                acc = acc + d * d
        return acc

    tv0 = fire(0, 0)
    fcopy.wait()

    def wave_step(w, carry):
        tvp, acc = carry
        buf = lax.rem(w, 2)
        tv = fire(w, buf)
        pbuf = lax.rem(w + 1, 2)
        drain(pbuf)
        acc = compute(w - 1, pbuf, tvp, acc)
        return tv, acc

    tvl, acc = lax.fori_loop(
        1, _NWAVE, wave_step, (tv0, jnp.zeros((_LANES,), jnp.float32)))
    last = _NWAVE - 1
    lbuf = last % 2
    drain(lbuf)
    acc = compute(last, lbuf, tvl, acc)

    res_v[...] = acc * _SCALE
    pltpu.sync_copy(res_v, out_hbm.at[wid])


@jax.jit
def kernel(features, targets, centers):
    idx = targets.astype(jnp.int32)
    centers2 = _pack(jnp.transpose(centers), _CGRID, _CROWS)
    feat2 = _pack(jnp.transpose(features), _B // (2 * _CB), _FROWS)
    run = functools.partial(
        pl.kernel,
        out_type=jax.ShapeDtypeStruct((_NW, _LANES), jnp.float32),
        mesh=plsc.VectorSubcoreMesh(core_axis_name="c", subcore_axis_name="s"),
        scratch_types=[
            pltpu.VMEM((_BPW,), jnp.int32),
            pltpu.VMEM((2, _W, 2 * _D), jnp.float32),
            pltpu.VMEM((_BPW, 2 * _D), jnp.float32),
            pltpu.VMEM((_LANES,), jnp.float32),
            pltpu.SemaphoreType.DMA((2,)),
            pltpu.SemaphoreType.DMA,
        ],
    )(_body)
    partials = run(idx, feat2, centers2)
    return jnp.sum(partials)


# speedup vs baseline: 3.1216x; 1.1342x over previous
import functools

import jax
import jax.numpy as jnp
from jax import lax
from jax.experimental import pallas as pl
from jax.experimental.pallas import tpu as pltpu
from jax.experimental.pallas import tpu_sc as plsc

_NUM_CLASSES = 1000000
_D = 64
_B = 16384
_LAMBDA_C = 0.5

_NC = 2
_NS = 16
_NW = _NC * _NS
_BPW = _B // _NW
_W = 16
_NWAVE = _BPW // _W
_LANES = 16
_SCALE = _LAMBDA_C / (2.0 * _B)

_CB = 8192
_CSH = 13
_CGRID = (_NUM_CLASSES + 2 * _CB - 1) // (2 * _CB)
_CROWS = _CGRID * _CB
_FROWS = _B // 2


def _tpose_body(x1_ref, x2_ref, out_ref):
    z = jnp.concatenate([x1_ref[...], x2_ref[...]], axis=0)
    r = lax.broadcasted_iota(jnp.int32, (2 * _D, 2 * _D), 0)
    c = lax.broadcasted_iota(jnp.int32, (2 * _D, 2 * _D), 1)
    ident = (r == c).astype(jnp.float32)
    out_ref[...] = lax.dot_general(
        z, ident, (((0,), (0,)), ((), ())),
        preferred_element_type=jnp.float32)


def _pack(ct, ngrid, nrows):
    nblk = pl.cdiv(ct.shape[1], _CB)
    return pl.pallas_call(
        _tpose_body,
        grid=(ngrid,),
        in_specs=[
            pl.BlockSpec((_D, _CB), lambda i: (0, 2 * i)),
            pl.BlockSpec((_D, _CB), lambda i: (0, jnp.minimum(2 * i + 1,
                                                              nblk - 1))),
        ],
        out_specs=pl.BlockSpec((_CB, 2 * _D), lambda i: (i, 0)),
        out_shape=jax.ShapeDtypeStruct((nrows, 2 * _D), jnp.float32),
    )(ct, ct)


def _body(idx_hbm, feat_hbm, c2_hbm, out_hbm,
          idx_v, rows_v, feat_v, res_v, gsem, fsem):
    wid = lax.axis_index("s") * _NC + lax.axis_index("c")
    base = wid * _BPW

    pltpu.sync_copy(idx_hbm.at[pl.ds(base, _BPW)], idx_v)
    nw_grp = 2 * _CB // _BPW
    frow = pl.multiple_of(
        (wid // nw_grp) * _CB + (wid % (nw_grp // 2)) * _BPW, _BPW)
    fhalf = (wid // (nw_grp // 2)) & 1
    fcopy = pltpu.async_copy(feat_hbm.at[pl.ds(frow, _BPW)], feat_v, fsem)

    def fire(w, buf):
        tv = idx_v[pl.ds(w * _W, _W)]
        row = jnp.bitwise_or(
            lax.shift_left(lax.shift_right_logical(tv, _CSH + 1), _CSH),
            tv & (_CB - 1))
        pltpu.async_copy(c2_hbm.at[row], rows_v.at[buf], gsem.at[buf])
        return tv

    def drain(buf):
        pltpu.make_async_copy(
            c2_hbm.at[pl.ds(0, _W)],
            rows_v.at[buf],
            gsem.at[buf],
        ).wait()

    def compute(w, buf, tv, acc):
        for k in range(_W):
            w_hi = ((lax.shift_right_logical(tv[k], _CSH)) & 1).astype(
                jnp.float32)
            w_lo = 1.0 - w_hi
            foff = fhalf * _D
            for c in range(_D // _LANES):
                f = feat_v[w * _W + k, pl.ds(foff + c * _LANES, _LANES)]
                g_lo = rows_v[buf, k, pl.ds(c * _LANES, _LANES)]
                g_hi = rows_v[buf, k, pl.ds(_D + c * _LANES, _LANES)]
                d = f - (g_lo * w_lo + g_hi * w_hi)
                acc = acc + d * d
        return acc

    tv0 = fire(0, 0)
    fcopy.wait()

    def wave_step(w, carry):
        tvp, acc = carry
        buf = lax.rem(w, 2)
        tv = fire(w, buf)
        pbuf = lax.rem(w + 1, 2)
        drain(pbuf)
        acc = compute(w - 1, pbuf, tvp, acc)
        return tv, acc

    tvl, acc = lax.fori_loop(
        1, _NWAVE, wave_step, (tv0, jnp.zeros((_LANES,), jnp.float32)))
    last = _NWAVE - 1
    lbuf = last % 2
    drain(lbuf)
    acc = compute(last, lbuf, tvl, acc)

    res_v[...] = acc * _SCALE
    pltpu.sync_copy(res_v, out_hbm.at[wid])


@jax.jit
def kernel(features, targets, centers):
    idx = targets.astype(jnp.int32)
    centers2 = _pack(jnp.transpose(centers), _CGRID, _CROWS)
    feat2 = _pack(jnp.transpose(features), _B // (2 * _CB), _FROWS)
    run = functools.partial(
        pl.kernel,
        out_type=jax.ShapeDtypeStruct((_NW, _LANES), jnp.float32),
        mesh=plsc.VectorSubcoreMesh(core_axis_name="c", subcore_axis_name="s"),
        scratch_types=[
            pltpu.VMEM((_BPW,), jnp.int32),
            pltpu.VMEM((2, _W, 2 * _D), jnp.float32),
            pltpu.VMEM((_BPW, 2 * _D), jnp.float32),
            pltpu.VMEM((_LANES,), jnp.float32),
            pltpu.SemaphoreType.DMA((2,)),
            pltpu.SemaphoreType.DMA,
        ],
    )(_body)
    partials = run(idx, feat2, centers2)
    return jnp.sum(partials)


# speedup vs baseline: 3.1887x; 1.0215x over previous
import functools

import jax
import jax.numpy as jnp
from jax import lax
from jax.experimental import pallas as pl
from jax.experimental.pallas import tpu as pltpu
from jax.experimental.pallas import tpu_sc as plsc

_NUM_CLASSES = 1000000
_D = 64
_B = 16384
_LAMBDA_C = 0.5

_NC = 2
_NS = 16
_NW = _NC * _NS
_BPW = _B // _NW
_W = 16
_NWAVE = _BPW // _W
_LANES = 16
_SCALE = _LAMBDA_C / (2.0 * _B)

_CB = 16384
_CSH = 14
_FCB = 8192
_CGRID = (_NUM_CLASSES + 2 * _CB - 1) // (2 * _CB)
_CROWS = _CGRID * _CB
_FROWS = _B // 2


def _tpose_body(x1_ref, x2_ref, out_ref):
    z = jnp.concatenate([x1_ref[...], x2_ref[...]], axis=0)
    r = lax.broadcasted_iota(jnp.int32, (2 * _D, 2 * _D), 0)
    c = lax.broadcasted_iota(jnp.int32, (2 * _D, 2 * _D), 1)
    ident = (r == c).astype(jnp.float32)
    out_ref[...] = lax.dot_general(
        z, ident, (((0,), (0,)), ((), ())),
        preferred_element_type=jnp.float32)


def _pack(ct, ngrid, nrows, cb):
    nblk = pl.cdiv(ct.shape[1], cb)
    return pl.pallas_call(
        _tpose_body,
        grid=(ngrid,),
        in_specs=[
            pl.BlockSpec((_D, cb), lambda i: (0, 2 * i)),
            pl.BlockSpec((_D, cb), lambda i: (0, jnp.minimum(2 * i + 1,
                                                             nblk - 1))),
        ],
        out_specs=pl.BlockSpec((cb, 2 * _D), lambda i: (i, 0)),
        out_shape=jax.ShapeDtypeStruct((nrows, 2 * _D), jnp.float32),
    )(ct, ct)


def _body(idx_hbm, feat_hbm, c2_hbm, out_hbm,
          idx_v, rows_v, feat_v, res_v, gsem, fsem):
    wid = lax.axis_index("s") * _NC + lax.axis_index("c")
    base = wid * _BPW

    pltpu.sync_copy(idx_hbm.at[pl.ds(base, _BPW)], idx_v)
    nw_grp = 2 * _FCB // _BPW
    frow = pl.multiple_of(
        (wid // nw_grp) * _FCB + (wid % (nw_grp // 2)) * _BPW, _BPW)
    fhalf = (wid // (nw_grp // 2)) & 1
    fcopy = pltpu.async_copy(feat_hbm.at[pl.ds(frow, _BPW)], feat_v, fsem)

    def fire(w, buf):
        tv = idx_v[pl.ds(w * _W, _W)]
        row = jnp.bitwise_or(
            lax.shift_left(lax.shift_right_logical(tv, _CSH + 1), _CSH),
            tv & (_CB - 1))
        pltpu.async_copy(c2_hbm.at[row], rows_v.at[buf], gsem.at[buf])
        return tv

    def drain(buf):
        pltpu.make_async_copy(
            c2_hbm.at[pl.ds(0, _W)],
            rows_v.at[buf],
            gsem.at[buf],
        ).wait()

    def compute(w, buf, tv, acc):
        for k in range(_W):
            w_hi = ((lax.shift_right_logical(tv[k], _CSH)) & 1).astype(
                jnp.float32)
            w_lo = 1.0 - w_hi
            foff = fhalf * _D
            for c in range(_D // _LANES):
                f = feat_v[w * _W + k, pl.ds(foff + c * _LANES, _LANES)]
                g_lo = rows_v[buf, k, pl.ds(c * _LANES, _LANES)]
                g_hi = rows_v[buf, k, pl.ds(_D + c * _LANES, _LANES)]
                d = f - (g_lo * w_lo + g_hi * w_hi)
                acc = acc + d * d
        return acc

    tv0 = fire(0, 0)
    fcopy.wait()

    def wave_step(w, carry):
        tvp, acc = carry
        buf = lax.rem(w, 2)
        tv = fire(w, buf)
        pbuf = lax.rem(w + 1, 2)
        drain(pbuf)
        acc = compute(w - 1, pbuf, tvp, acc)
        return tv, acc

    tvl, acc = lax.fori_loop(
        1, _NWAVE, wave_step, (tv0, jnp.zeros((_LANES,), jnp.float32)))
    last = _NWAVE - 1
    lbuf = last % 2
    drain(lbuf)
    acc = compute(last, lbuf, tvl, acc)

    res_v[...] = acc * _SCALE
    pltpu.sync_copy(res_v, out_hbm.at[wid])


@jax.jit
def kernel(features, targets, centers):
    idx = targets.astype(jnp.int32)
    centers2 = _pack(jnp.transpose(centers), _CGRID, _CROWS, _CB)
    feat2 = _pack(jnp.transpose(features), _B // (2 * _FCB), _FROWS, _FCB)
    run = functools.partial(
        pl.kernel,
        out_type=jax.ShapeDtypeStruct((_NW, _LANES), jnp.float32),
        mesh=plsc.VectorSubcoreMesh(core_axis_name="c", subcore_axis_name="s"),
        scratch_types=[
            pltpu.VMEM((_BPW,), jnp.int32),
            pltpu.VMEM((2, _W, 2 * _D), jnp.float32),
            pltpu.VMEM((_BPW, 2 * _D), jnp.float32),
            pltpu.VMEM((_LANES,), jnp.float32),
            pltpu.SemaphoreType.DMA((2,)),
            pltpu.SemaphoreType.DMA,
        ],
    )(_body)
    partials = run(idx, feat2, centers2)
    return jnp.sum(partials)
